# baseline (speedup 1.0000x reference)
import functools

import jax
import jax.numpy as jnp
from jax import lax
from jax.experimental import pallas as pl
from jax.experimental.pallas import tpu as pltpu
from jax.experimental.pallas import tpu_sc as plsc

_NUM_ORDERS = 3
_FEATURES = 128
_NUM_EMB = 100000
_NUM_TABLES = 8
_SHARD = _FEATURES // _NUM_TABLES
_MULT = 1000003
_PRIMES = (2, 3, 5, 7, 11, 13, 17, 19)

_B = 1024
_T = 200
_L = 16
_NC = 2
_NS = 16
_NW = _NC * _NS
_ROWS_PER_WORKER = _B // _NW
_GROUPS = 13
_TPAD = _GROUPS * _L
_NTAB = _NUM_ORDERS * _NUM_TABLES
_RINV = 1.0 / _NUM_EMB


def _floor_mod_1e5(v):
    q = (v.astype(jnp.float32) * _RINV).astype(jnp.int32)
    r = v - q * _NUM_EMB
    r = jnp.where(r < 0, r + _NUM_EMB, r)
    r = jnp.where(r < 0, r + _NUM_EMB, r)
    r = jnp.where(r >= _NUM_EMB, r - _NUM_EMB, r)
    return r


def _sc_body(ids_hbm, table_hbm, out_hbm,
             ids_all, idx_v, buf, stage, sem):
    wid = lax.axis_index("c") * _NS + lax.axis_index("s")
    r_base = wid * _ROWS_PER_WORKER
    pltpu.sync_copy(ids_hbm.at[pl.ds(r_base, _ROWS_PER_WORKER)], ids_all)

    def hash_row(k, slot_idx):
        row = ids_all.at[k]

        def grp(g, c2):
            t0 = pl.multiple_of(g * _L, _L)
            a = row[pl.ds(t0, _L)]
            b = row[pl.ds(t0 + 1, _L)]
            c = row[pl.ds(t0 + 2, _L)]
            fp2 = a * _MULT + b
            fp3 = fp2 * _MULT + c
            for oi, fp in enumerate((a, fp2, fp3)):
                fpp = fp + 1
                for ti in range(_NUM_TABLES):
                    h = _floor_mod_1e5(fpp * _PRIMES[ti])
                    slot_idx[oi * _NUM_TABLES + ti, pl.ds(t0, _L)] = h
            return c2

        lax.fori_loop(0, _GROUPS, grp, 0)

    def gather_row(slot_idx):
        return [pltpu.async_copy(
                    table_hbm.at[j].at[slot_idx.at[j]],
                    buf.at[pl.ds(j * _TPAD, _TPAD)], sem)
                for j in range(_NTAB)]

    def acc(r):
        def one_token(t, c3):
            for ti in range(_NUM_TABLES):
                base = ti * _TPAD + t
                s = (buf[base, :] + buf[_NUM_TABLES * _TPAD + base, :]
                     + buf[2 * _NUM_TABLES * _TPAD + base, :])
                stage[t, pl.ds(ti * _SHARD, _SHARD)] = s
            return c3

        lax.fori_loop(0, _T, one_token, 0)

    def store(r):
        pltpu.sync_copy(stage, out_hbm.at[pl.ds(r * _T, _T)])

    hash_row(0, idx_v.at[0])

    def row_pair(k, carry):
        k0 = 2 * k
        cps0 = gather_row(idx_v.at[0])
        hash_row(k0 + 1, idx_v.at[1])
        for cp in cps0:
            cp.wait()
        acc(r_base + k0)
        cps1 = gather_row(idx_v.at[1])
        store(r_base + k0)
        hash_row(jnp.minimum(k0 + 2, _ROWS_PER_WORKER - 1), idx_v.at[0])
        for cp in cps1:
            cp.wait()
        acc(r_base + k0 + 1)
        store(r_base + k0 + 1)
        return carry

    lax.fori_loop(0, _ROWS_PER_WORKER // 2, row_pair, 0)


@jax.jit
def _ngram_embed_sc(input_ids, table3):
    mesh = plsc.VectorSubcoreMesh(core_axis_name="c", subcore_axis_name="s")
    fn = functools.partial(
        pl.kernel,
        out_type=jax.ShapeDtypeStruct((_B * _T, _FEATURES), jnp.float32),
        mesh=mesh,
        compiler_params=pltpu.CompilerParams(
            needs_layout_passes=False, use_tc_tiling_on_sc=False),
        scratch_types=[
            pltpu.VMEM((_ROWS_PER_WORKER, 256), jnp.int32),
            pltpu.VMEM((2, _NTAB, _TPAD), jnp.int32),
            pltpu.VMEM((_NTAB * _TPAD, _SHARD), jnp.float32),
            pltpu.VMEM((_T, _FEATURES), jnp.float32),
            pltpu.SemaphoreType.DMA,
        ],
    )(_sc_body)
    return fn(input_ids, table3)


def kernel(input_ids, tables):
    table3 = tables.reshape(_NTAB, _NUM_EMB, _SHARD)
    ids_pad = jnp.zeros((_B, 256), jnp.int32).at[:, :_T].set(
        input_ids.astype(jnp.int32))
    out = _ngram_embed_sc(ids_pad, table3)
    return out.reshape(_B, _T, _FEATURES)


# speedup vs baseline: 1.1318x; 1.1318x over previous
import functools

import jax
import jax.numpy as jnp
from jax import lax
from jax.experimental import pallas as pl
from jax.experimental.pallas import tpu as pltpu
from jax.experimental.pallas import tpu_sc as plsc

_NUM_ORDERS = 3
_FEATURES = 128
_NUM_EMB = 100000
_NUM_TABLES = 8
_SHARD = _FEATURES // _NUM_TABLES
_MULT = 1000003
_PRIMES = (2, 3, 5, 7, 11, 13, 17, 19)

_B = 1024
_T = 200
_L = 16
_NC = 2
_NS = 16
_NW = _NC * _NS
_ROWS_PER_WORKER = _B // _NW
_GROUPS = 13
_TPAD = _GROUPS * _L
_RPC = _TPAD * _NUM_TABLES
_ROWS_OUT = _T * _NUM_TABLES
_NIDX = _NUM_ORDERS * _RPC
_RINV = 1.0 / _NUM_EMB


def _floor_mod_1e5(v):
    q = (v.astype(jnp.float32) * _RINV).astype(jnp.int32)
    r = v - q * _NUM_EMB
    r = jnp.where(r < 0, r + _NUM_EMB, r)
    r = jnp.where(r < 0, r + _NUM_EMB, r)
    r = jnp.where(r >= _NUM_EMB, r - _NUM_EMB, r)
    return r


def _sc_body(ids_hbm, table_hbm, out_hbm,
             ids_all, idx_v, buf, sem):
    wid = lax.axis_index("c") * _NS + lax.axis_index("s")
    iota = lax.iota(jnp.int32, _L)
    r_base = wid * _ROWS_PER_WORKER
    pltpu.sync_copy(ids_hbm.at[pl.ds(r_base, _ROWS_PER_WORKER)], ids_all)

    def hash_row(k, slot_idx):
        row = ids_all.at[k]

        def grp(g, c2):
            t0 = pl.multiple_of(g * _L, _L)
            a = row[pl.ds(t0, _L)]
            b = row[pl.ds(t0 + 1, _L)]
            c = row[pl.ds(t0 + 2, _L)]
            fp2 = a * _MULT + b
            fp3 = fp2 * _MULT + c
            col = iota * _NUM_TABLES
            for oi, fp in enumerate((a, fp2, fp3)):
                fpp = fp + 1
                base = col + (oi * _GROUPS + g) * 128
                for ti in range(_NUM_TABLES):
                    h = _floor_mod_1e5(fpp * _PRIMES[ti])
                    h = h + ((oi * _NUM_TABLES + ti) * _NUM_EMB)
                    plsc.store_scatter(slot_idx, [base + ti], h)
            return c2

        lax.fori_loop(0, _GROUPS, grp, 0)

    def acc_store(r):
        def acc(i, c3):
            i2 = i * 2
            buf[i2, :] = (buf[i2, :] + buf[_RPC + i2, :]
                          + buf[2 * _RPC + i2, :])
            buf[i2 + 1, :] = (buf[i2 + 1, :] + buf[_RPC + i2 + 1, :]
                              + buf[2 * _RPC + i2 + 1, :])
            return c3

        lax.fori_loop(0, _ROWS_OUT // 2, acc, 0)
        pltpu.sync_copy(buf.at[pl.ds(0, _ROWS_OUT)],
                        out_hbm.at[pl.ds(r * _ROWS_OUT, _ROWS_OUT)])

    hash_row(0, idx_v.at[0])

    def row_pair(k, carry):
        k0 = 2 * k
        cp0 = pltpu.async_copy(table_hbm.at[idx_v.at[0]], buf, sem)
        hash_row(k0 + 1, idx_v.at[1])
        cp0.wait()
        acc_store(r_base + k0)
        cp1 = pltpu.async_copy(table_hbm.at[idx_v.at[1]], buf, sem)
        hash_row(jnp.minimum(k0 + 2, _ROWS_PER_WORKER - 1), idx_v.at[0])
        cp1.wait()
        acc_store(r_base + k0 + 1)
        return carry

    lax.fori_loop(0, _ROWS_PER_WORKER // 2, row_pair, 0)


@jax.jit
def _ngram_embed_sc(input_ids, table_flat):
    mesh = plsc.VectorSubcoreMesh(core_axis_name="c", subcore_axis_name="s")
    fn = functools.partial(
        pl.kernel,
        out_type=jax.ShapeDtypeStruct((_B * _ROWS_OUT, _SHARD), jnp.float32),
        mesh=mesh,
        compiler_params=pltpu.CompilerParams(
            needs_layout_passes=False, use_tc_tiling_on_sc=False),
        scratch_types=[
            pltpu.VMEM((_ROWS_PER_WORKER, 256), jnp.int32),
            pltpu.VMEM((2, _NIDX), jnp.int32),
            pltpu.VMEM((_NIDX, _SHARD), jnp.float32),
            pltpu.SemaphoreType.DMA,
        ],
    )(_sc_body)
    return fn(input_ids, table_flat)


def kernel(input_ids, tables):
    table_flat = tables.reshape(_NUM_ORDERS * _NUM_TABLES * _NUM_EMB, _SHARD)
    ids_pad = jnp.zeros((_B, 256), jnp.int32).at[:, :_T].set(
        input_ids.astype(jnp.int32))
    out = _ngram_embed_sc(ids_pad, table_flat)
    return out.reshape(_B, _T, _FEATURES)


# speedup vs baseline: 1.1796x; 1.0422x over previous
import functools

import jax
import jax.numpy as jnp
from jax import lax
from jax.experimental import pallas as pl
from jax.experimental.pallas import tpu as pltpu
from jax.experimental.pallas import tpu_sc as plsc

_NUM_ORDERS = 3
_FEATURES = 128
_NUM_EMB = 100000
_NUM_TABLES = 8
_SHARD = _FEATURES // _NUM_TABLES
_MULT = 1000003
_PRIMES = (2, 3, 5, 7, 11, 13, 17, 19)

_B = 1024
_T = 200
_L = 16
_NC = 2
_NS = 16
_NW = _NC * _NS
_ROWS_PER_WORKER = _B // _NW
_GROUPS = 13
_TPAD = _GROUPS * _L
_RPC = _TPAD * _NUM_TABLES
_ROWS_OUT = _T * _NUM_TABLES
_NIDX = _NUM_ORDERS * _RPC
_RINV = 1.0 / _NUM_EMB


def _floor_mod_1e5(v):
    q = (v.astype(jnp.float32) * _RINV).astype(jnp.int32)
    r = v - q * _NUM_EMB
    r = jnp.where(r < 0, r + _NUM_EMB, r)
    r = jnp.where(r < 0, r + _NUM_EMB, r)
    r = jnp.where(r >= _NUM_EMB, r - _NUM_EMB, r)
    return r


def _sc_body(ids_hbm, table_hbm, out_hbm,
             ids_all, idx_v, buf, sem):
    wid = lax.axis_index("c") * _NS + lax.axis_index("s")
    iota = lax.iota(jnp.int32, _L)
    r_base = wid * _ROWS_PER_WORKER
    pltpu.sync_copy(ids_hbm.at[pl.ds(r_base, _ROWS_PER_WORKER)], ids_all)

    def hash_row(k, slot_idx):
        row = ids_all.at[k]

        def grp(g, c2):
            t0 = pl.multiple_of(g * _L, _L)
            a = row[pl.ds(t0, _L)]
            b = row[pl.ds(t0 + 1, _L)]
            c = row[pl.ds(t0 + 2, _L)]
            fp2 = a * _MULT + b
            fp3 = fp2 * _MULT + c
            col = iota * _NUM_TABLES
            for oi, fp in enumerate((a, fp2, fp3)):
                fpp = fp + 1
                base = col + (oi * _GROUPS + g) * 128
                for ti in range(_NUM_TABLES):
                    h = _floor_mod_1e5(fpp * _PRIMES[ti])
                    h = h + ((oi * _NUM_TABLES + ti) * _NUM_EMB)
                    plsc.store_scatter(slot_idx, [base + ti], h)
            return c2

        lax.fori_loop(0, _GROUPS, grp, 0)

    def acc_store(r):
        def acc(i, c3):
            i2 = i * 8
            for u in range(8):
                buf[i2 + u, :] = (buf[i2 + u, :] + buf[_RPC + i2 + u, :]
                                  + buf[2 * _RPC + i2 + u, :])
            return c3

        lax.fori_loop(0, _ROWS_OUT // 8, acc, 0)
        pltpu.sync_copy(buf.at[pl.ds(0, _ROWS_OUT)],
                        out_hbm.at[pl.ds(r * _ROWS_OUT, _ROWS_OUT)])

    hash_row(0, idx_v.at[0])

    def row_pair(k, carry):
        k0 = 2 * k
        cp0 = pltpu.async_copy(table_hbm.at[idx_v.at[0]], buf, sem)
        hash_row(k0 + 1, idx_v.at[1])
        cp0.wait()
        acc_store(r_base + k0)
        cp1 = pltpu.async_copy(table_hbm.at[idx_v.at[1]], buf, sem)
        hash_row(jnp.minimum(k0 + 2, _ROWS_PER_WORKER - 1), idx_v.at[0])
        cp1.wait()
        acc_store(r_base + k0 + 1)
        return carry

    lax.fori_loop(0, _ROWS_PER_WORKER // 2, row_pair, 0)


@jax.jit
def _ngram_embed_sc(input_ids, table_flat):
    mesh = plsc.VectorSubcoreMesh(core_axis_name="c", subcore_axis_name="s")
    fn = functools.partial(
        pl.kernel,
        out_type=jax.ShapeDtypeStruct((_B * _ROWS_OUT, _SHARD), jnp.float32),
        mesh=mesh,
        compiler_params=pltpu.CompilerParams(
            needs_layout_passes=False, use_tc_tiling_on_sc=False),
        scratch_types=[
            pltpu.VMEM((_ROWS_PER_WORKER, 256), jnp.int32),
            pltpu.VMEM((2, _NIDX), jnp.int32),
            pltpu.VMEM((_NIDX, _SHARD), jnp.float32),
            pltpu.SemaphoreType.DMA,
        ],
    )(_sc_body)
    return fn(input_ids, table_flat)


def kernel(input_ids, tables):
    table_flat = tables.reshape(_NUM_ORDERS * _NUM_TABLES * _NUM_EMB, _SHARD)
    ids_pad = jnp.zeros((_B, 256), jnp.int32).at[:, :_T].set(
        input_ids.astype(jnp.int32))
    out = _ngram_embed_sc(ids_pad, table_flat)
    return out.reshape(_B, _T, _FEATURES)
